# Initial kernel scaffold; baseline (speedup 1.0000x reference)
import functools

import jax
import jax.numpy as jnp
from jax.experimental import pallas as pl
from jax.experimental.pallas import tpu as pltpu

_N = 4096
_D = 16
_BLK = 256
_G = _N // _BLK
_NCLUST = 128


def _body(e_ref, labf_ref, labc_ref, out_ref, en_ref, acc_ref, nu_ref):
    i = pl.program_id(0)

    @pl.when(i == 0)
    def _init():
        e = e_ref[...]
        ss = jnp.sum(e * e, axis=1, keepdims=True)
        inv = jax.lax.rsqrt(jnp.maximum(ss, 1e-24))
        en_ref[...] = e * inv
        ids = jax.lax.broadcasted_iota(jnp.int32, (_NCLUST, _N), 0)
        present = jnp.any(labf_ref[...] == ids, axis=1)
        nu_ref[0, 0] = jnp.sum(present.astype(jnp.float32))
        acc_ref[0, 0] = 0.0

    en = en_ref[...]
    rows = en_ref[pl.ds(i * _BLK, _BLK), :]
    g = jax.lax.dot_general(rows, en, (((1,), (1,)), ((), ())),
                            preferred_element_type=jnp.float32)
    sqr = jnp.sum(rows * rows, axis=1, keepdims=True)
    sqa = jnp.sum(en * en, axis=1)[None, :]
    sq = jnp.maximum(sqr + sqa - 2.0 * g, 0.0)
    dist = jnp.sqrt(sq)
    mask = labc_ref[...] == labf_ref[...]
    acc_ref[0, 0] += jnp.sum(jnp.where(mask, dist, 0.0))

    @pl.when(i == _G - 1)
    def _fin():
        out_ref[0, 0] = acc_ref[0, 0] / nu_ref[0, 0]


def kernel(embeddings, cluster_labels):
    labels = cluster_labels.astype(jnp.int32)
    out = pl.pallas_call(
        _body,
        grid=(_G,),
        in_specs=[
            pl.BlockSpec((_N, _D), lambda i: (0, 0)),
            pl.BlockSpec((1, _N), lambda i: (0, 0)),
            pl.BlockSpec((_BLK, 1), lambda i: (i, 0)),
        ],
        out_specs=pl.BlockSpec((1, 1), lambda i: (0, 0)),
        out_shape=jax.ShapeDtypeStruct((1, 1), jnp.float32),
        scratch_shapes=[
            pltpu.VMEM((_N, _D), jnp.float32),
            pltpu.SMEM((1, 1), jnp.float32),
            pltpu.SMEM((1, 1), jnp.float32),
        ],
    )(embeddings, labels.reshape(1, _N), labels.reshape(_N, 1))
    return out[0, 0]


# speedup vs baseline: 1.1892x; 1.1892x over previous
import functools

import jax
import jax.numpy as jnp
from jax.experimental import pallas as pl
from jax.experimental.pallas import tpu as pltpu

_N = 4096
_D = 16
_BLK = 256
_G = _N // _BLK
_NCLUST = 128


def _body(e_ref, labf_ref, labc_ref, out_ref, en_ref, acc_ref, nu_ref):
    i = pl.program_id(0)

    @pl.when(i == 0)
    def _init():
        e = e_ref[...]
        ss = jnp.sum(e * e, axis=1, keepdims=True)
        inv = jax.lax.rsqrt(jnp.maximum(ss, 1e-24))
        en_ref[...] = e * inv
        ids = jax.lax.broadcasted_iota(jnp.int32, (_NCLUST, _N), 0)
        present = jnp.any(labf_ref[...] == ids, axis=1)
        nu_ref[0, 0] = jnp.sum(present.astype(jnp.float32))
        acc_ref[0, 0] = 0.0

    en = en_ref[...]
    rows = en_ref[pl.ds(i * _BLK, _BLK), :]
    g = jax.lax.dot_general(rows, en, (((1,), (1,)), ((), ())),
                            preferred_element_type=jnp.float32)
    sqr = jnp.sum(rows * rows, axis=1, keepdims=True)
    sqa = jnp.sum(en * en, axis=1)[None, :]
    sq = jnp.maximum(sqr + sqa - 2.0 * g, 0.0)
    dist = jnp.sqrt(sq)
    mask = labc_ref[...] == labf_ref[...]
    acc_ref[0, 0] += jnp.sum(jnp.where(mask, dist, 0.0))

    @pl.when(i == _G - 1)
    def _fin():
        out_ref[...] = jnp.full((1, 1), acc_ref[0, 0] / nu_ref[0, 0],
                                dtype=jnp.float32)


def kernel(embeddings, cluster_labels):
    labels = cluster_labels.astype(jnp.int32)
    out = pl.pallas_call(
        _body,
        grid=(_G,),
        in_specs=[
            pl.BlockSpec((_N, _D), lambda i: (0, 0)),
            pl.BlockSpec((1, _N), lambda i: (0, 0)),
            pl.BlockSpec((_BLK, 1), lambda i: (i, 0)),
        ],
        out_specs=pl.BlockSpec((1, 1), lambda i: (0, 0)),
        out_shape=jax.ShapeDtypeStruct((1, 1), jnp.float32),
        scratch_shapes=[
            pltpu.VMEM((_N, _D), jnp.float32),
            pltpu.SMEM((1, 1), jnp.float32),
            pltpu.SMEM((1, 1), jnp.float32),
        ],
    )(embeddings, labels.reshape(1, _N), labels.reshape(_N, 1))
    return out[0, 0]
